# Initial kernel scaffold; baseline (speedup 1.0000x reference)
import jax
import jax.numpy as jnp
from jax.experimental import pallas as pl


def kernel(x, W1, b1, W2, b2, Ws, bs, add_s, sub_s, mul_s, div_s):
    raise NotImplementedError("write your pallas kernel here")


# speedup vs baseline: 5.0897x; 5.0897x over previous
import functools

import jax
import jax.numpy as jnp
from jax.experimental import pallas as pl
from jax.experimental.pallas import tpu as pltpu

_S = 4
_M = 6


def _xor_shuffle(v, j, n, lane):
    if j >= 128:
        m = j // 128
        cc = n // 128
        v5 = v.reshape(v.shape[0], cc // (2 * m), 2, m, 128)
        sw = jnp.concatenate([v5[:, :, 1:2], v5[:, :, 0:1]], axis=2)
        return sw.reshape(v.shape[0], n)
    low = (lane & j) == 0
    return jnp.where(low, pltpu.roll(v, n - j, axis=1), pltpu.roll(v, j, axis=1))


def _bitonic_sort_rows(v):
    r, n = v.shape
    lane = jax.lax.broadcasted_iota(jnp.int32, (r, n), 1)
    k = 2
    while k <= n:
        j = k // 2
        while j >= 1:
            p = _xor_shuffle(v, j, n, lane)
            up = (lane & k) == 0
            lowhalf = (lane & j) == 0
            take_min = lowhalf == up
            v = jnp.where(take_min, jnp.minimum(v, p), jnp.maximum(v, p))
            j //= 2
        k *= 2
    return v


def _flip_rows(v):
    r, n = v.shape
    cc = n // 128
    v3 = v.reshape(r, cc, 128)
    v3 = jnp.concatenate([v3[:, i : i + 1, :] for i in range(cc - 1, -1, -1)], axis=1)
    lane = jax.lax.broadcasted_iota(jnp.int32, (r, cc, 128), 2)
    out = v3
    j = 64
    while j >= 1:
        low = (lane & j) == 0
        out = jnp.where(low, pltpu.roll(out, 128 - j, axis=2), pltpu.roll(out, j, axis=2))
        j //= 2
    return out.reshape(r, n)


def _bf16_dot(a, b_t):
    return jax.lax.dot_general(
        a.astype(jnp.bfloat16),
        b_t.astype(jnp.bfloat16),
        (((1,), (1,)), ((), ())),
        preferred_element_type=jnp.float32,
    )


def _fused_kernel(x_ref, w1_ref, b1_ref, w2_ref, b2_ref, ws_ref, bs_ref, c_ref,
                  out_ref, lg_ref):
    x = x_ref[...]
    r = x.shape[0]
    d = x.shape[1]

    h = _bf16_dot(x, w1_ref[...]) + b1_ref[...]
    h = jnp.maximum(h, 0.0)
    enc = _bf16_dot(h, w2_ref[...]) + b2_ref[...]
    lg = _bf16_dot(enc, ws_ref[...]) + bs_ref[...]
    lg_ref[...] = lg

    add_s = c_ref[0, 0]
    sub_s = c_ref[0, 1]
    mul_s = c_ref[0, 2]
    div_s = c_ref[0, 3]
    inv_div = 1.0 / (div_s + 1e-05)

    col = jax.lax.broadcasted_iota(jnp.int32, (r, _S * _M), 1)
    a = jnp.ones((r, 1), jnp.float32)
    b = jnp.zeros((r, 1), jnp.float32)
    srt = jnp.zeros((r, 1), jnp.float32)
    flp = jnp.zeros((r, 1), jnp.float32)
    step_abs = []
    for s in range(_S):
        mask = (col >= s * _M) & (col < (s + 1) * _M)
        ml = jnp.where(mask, lg, -jnp.inf)
        mx = jnp.max(ml, axis=1, keepdims=True)
        idx = jnp.min(
            jnp.where(mask & (lg == mx), col - s * _M, _M), axis=1, keepdims=True
        )
        idx_f = idx.astype(jnp.float32)
        ind = [jnp.maximum(1.0 - jnp.abs(idx_f - k), 0.0) for k in range(_M)]
        a_op = 1.0 + ind[4] * (mul_s - 1.0) + ind[5] * (inv_div - 1.0)
        b_op = ind[2] * add_s - ind[3] * sub_s
        a_neg = jnp.maximum(jnp.sign(-a), 0.0)
        flp = (ind[0] * a_neg + ind[1] * (1.0 - flp)
               + (1.0 - ind[0] - ind[1]) * flp)
        srt = jnp.maximum(srt, ind[0])
        a = a * a_op
        b = b * a_op + b_op
        step_abs.append((a, b, srt, flp))

    sx = _bitonic_sort_rows(x)
    fx = _flip_rows(x)
    fsx = _flip_rows(sx)

    for s in range(_S):
        a, b, srt, flp = step_abs[s]
        base = (srt * (flp * fsx + (1.0 - flp) * sx)
                + (1.0 - srt) * (flp * fx + (1.0 - flp) * x))
        out_ref[:, s * d : (s + 1) * d] = a * base + b


@jax.jit
def kernel(x, W1, b1, W2, b2, Ws, bs, add_s, sub_s, mul_s, div_s):
    B, D = x.shape
    H = W1.shape[0]
    BR = 256
    consts = jnp.stack([add_s, sub_s, mul_s, div_s]).reshape(1, 4).astype(jnp.float32)
    out, lg = pl.pallas_call(
        _fused_kernel,
        grid=(B // BR,),
        in_specs=[
            pl.BlockSpec((BR, D), lambda i: (i, 0)),
            pl.BlockSpec((H, D), lambda i: (0, 0)),
            pl.BlockSpec((1, H), lambda i: (0, 0)),
            pl.BlockSpec((H, H), lambda i: (0, 0)),
            pl.BlockSpec((1, H), lambda i: (0, 0)),
            pl.BlockSpec((_S * _M, H), lambda i: (0, 0)),
            pl.BlockSpec((1, _S * _M), lambda i: (0, 0)),
            pl.BlockSpec((1, 4), lambda i: (0, 0)),
        ],
        out_specs=[
            pl.BlockSpec((BR, _S * D), lambda i: (i, 0)),
            pl.BlockSpec((BR, _S * _M), lambda i: (i, 0)),
        ],
        out_shape=[
            jax.ShapeDtypeStruct((B, _S * D), jnp.float32),
            jax.ShapeDtypeStruct((B, _S * _M), jnp.float32),
        ],
        compiler_params=pltpu.CompilerParams(
            dimension_semantics=("arbitrary",),
        ),
    )(x, W1, b1.reshape(1, H), W2, b2.reshape(1, H), Ws, bs.reshape(1, _S * _M),
      consts)
    return out.reshape(B, _S, D), lg.reshape(B, _S, _M)


# speedup vs baseline: 5.0982x; 1.0017x over previous
import functools

import jax
import jax.numpy as jnp
from jax.experimental import pallas as pl
from jax.experimental.pallas import tpu as pltpu

_S = 4
_M = 6


def _xor_shuffle(v, j, n, low):
    if j >= 128:
        m = j // 128
        cc = n // 128
        v5 = v.reshape(v.shape[0], cc // (2 * m), 2, m, 128)
        sw = jnp.concatenate([v5[:, :, 1:2], v5[:, :, 0:1]], axis=2)
        return sw.reshape(v.shape[0], n)
    return jnp.where(low, pltpu.roll(v, n - j, axis=1), pltpu.roll(v, j, axis=1))


def _bitonic_sort_rows(v, lane):
    r, n = v.shape
    k = 2
    while k <= n:
        up = (lane & k) == 0 if k < n else None
        j = k // 2
        while j >= 1:
            lowhalf = (lane & j) == 0
            p = _xor_shuffle(v, j, n, lowhalf)
            take_min = lowhalf if up is None else lowhalf == up
            v = jnp.where(take_min, jnp.minimum(v, p), jnp.maximum(v, p))
            j //= 2
        k *= 2
    return v


def _flip_rows(v):
    r, n = v.shape
    cc = n // 128
    v3 = v.reshape(r, cc, 128)
    v3 = jnp.concatenate([v3[:, i : i + 1, :] for i in range(cc - 1, -1, -1)], axis=1)
    lane = jax.lax.broadcasted_iota(jnp.int32, (r, cc, 128), 2)
    out = v3
    j = 64
    while j >= 1:
        low = (lane & j) == 0
        out = jnp.where(low, pltpu.roll(out, 128 - j, axis=2), pltpu.roll(out, j, axis=2))
        j //= 2
    return out.reshape(r, n)


def _bf16_dot(a, b_t):
    return jax.lax.dot_general(
        a.astype(jnp.bfloat16),
        b_t.astype(jnp.bfloat16),
        (((1,), (1,)), ((), ())),
        preferred_element_type=jnp.float32,
    )


def _fused_kernel(x_ref, w1_ref, b1_ref, w2_ref, b2_ref, ws_ref, bs_ref, c_ref,
                  out_ref, lg_ref):
    x = x_ref[...]
    r = x.shape[0]
    d = x.shape[1]

    h = _bf16_dot(x, w1_ref[...]) + b1_ref[...]
    h = jnp.maximum(h, 0.0)
    enc = _bf16_dot(h, w2_ref[...]) + b2_ref[...]
    lg = _bf16_dot(enc, ws_ref[...]) + bs_ref[...]
    lg_ref[...] = lg

    add_s = c_ref[0, 0]
    sub_s = c_ref[0, 1]
    mul_s = c_ref[0, 2]
    div_s = c_ref[0, 3]
    inv_div = 1.0 / (div_s + 1e-05)

    col = jax.lax.broadcasted_iota(jnp.int32, (r, _S * _M), 1)
    a = jnp.ones((r, 1), jnp.float32)
    b = jnp.zeros((r, 1), jnp.float32)
    srt = jnp.zeros((r, 1), jnp.float32)
    flp = jnp.zeros((r, 1), jnp.float32)
    step_abs = []
    for s in range(_S):
        mask = (col >= s * _M) & (col < (s + 1) * _M)
        ml = jnp.where(mask, lg, -jnp.inf)
        mx = jnp.max(ml, axis=1, keepdims=True)
        idx = jnp.min(
            jnp.where(mask & (lg == mx), col - s * _M, _M), axis=1, keepdims=True
        )
        idx_f = idx.astype(jnp.float32)
        ind = [jnp.maximum(1.0 - jnp.abs(idx_f - k), 0.0) for k in range(_M)]
        a_op = 1.0 + ind[4] * (mul_s - 1.0) + ind[5] * (inv_div - 1.0)
        b_op = ind[2] * add_s - ind[3] * sub_s
        a_neg = jnp.maximum(jnp.sign(-a), 0.0)
        flp = (ind[0] * a_neg + ind[1] * (1.0 - flp)
               + (1.0 - ind[0] - ind[1]) * flp)
        srt = jnp.maximum(srt, ind[0])
        a = a * a_op
        b = b * a_op + b_op
        step_abs.append((a, b, srt, flp))

    lane = jax.lax.broadcasted_iota(jnp.int32, (r, d), 1)
    sx = _bitonic_sort_rows(x, lane)
    fx = _flip_rows(x)
    fsx = _flip_rows(sx)

    for s in range(_S):
        a, b, srt, flp = step_abs[s]
        base = (srt * (flp * fsx + (1.0 - flp) * sx)
                + (1.0 - srt) * (flp * fx + (1.0 - flp) * x))
        out_ref[:, s * d : (s + 1) * d] = a * base + b


@jax.jit
def kernel(x, W1, b1, W2, b2, Ws, bs, add_s, sub_s, mul_s, div_s):
    B, D = x.shape
    H = W1.shape[0]
    BR = 256
    consts = jnp.stack([add_s, sub_s, mul_s, div_s]).reshape(1, 4).astype(jnp.float32)
    out, lg = pl.pallas_call(
        _fused_kernel,
        grid=(B // BR,),
        in_specs=[
            pl.BlockSpec((BR, D), lambda i: (i, 0)),
            pl.BlockSpec((H, D), lambda i: (0, 0)),
            pl.BlockSpec((1, H), lambda i: (0, 0)),
            pl.BlockSpec((H, H), lambda i: (0, 0)),
            pl.BlockSpec((1, H), lambda i: (0, 0)),
            pl.BlockSpec((_S * _M, H), lambda i: (0, 0)),
            pl.BlockSpec((1, _S * _M), lambda i: (0, 0)),
            pl.BlockSpec((1, 4), lambda i: (0, 0)),
        ],
        out_specs=[
            pl.BlockSpec((BR, _S * D), lambda i: (i, 0)),
            pl.BlockSpec((BR, _S * _M), lambda i: (i, 0)),
        ],
        out_shape=[
            jax.ShapeDtypeStruct((B, _S * D), jnp.float32),
            jax.ShapeDtypeStruct((B, _S * _M), jnp.float32),
        ],
        compiler_params=pltpu.CompilerParams(
            dimension_semantics=("arbitrary",),
        ),
    )(x, W1, b1.reshape(1, H), W2, b2.reshape(1, H), Ws, bs.reshape(1, _S * _M),
      consts)
    return out.reshape(B, _S, D), lg.reshape(B, _S, _M)


# speedup vs baseline: 5.2817x; 1.0360x over previous
import functools

import jax
import jax.numpy as jnp
from jax.experimental import pallas as pl
from jax.experimental.pallas import tpu as pltpu

_S = 4
_M = 6


def _bitonic_sort_rows(v):
    r, n = v.shape
    cc = n // 128
    t = v.reshape(r * cc, 128)
    i0 = jax.lax.broadcasted_iota(jnp.int32, (r * cc, 128), 0)
    lane = jax.lax.broadcasted_iota(jnp.int32, (r * cc, 128), 1)
    k = 2
    while k <= n:
        s = None
        if k < n:
            if k <= 64:
                bit = lane & k
                s = 1.0 - bit.astype(jnp.float32) * (2.0 / k)
            else:
                m = k // 128
                bit = i0 & m
                s = 1.0 - bit.astype(jnp.float32) * (2.0 / m)
            t = t * s
        j = k // 2
        while j >= 1:
            if j >= 128:
                m = j // 128
                v5 = t.reshape(r, cc // (2 * m), 2, m, 128)
                p = jnp.concatenate(
                    [v5[:, :, 1:2], v5[:, :, 0:1]], axis=2
                ).reshape(r * cc, 128)
                low = (i0 & m) == 0
                t = jnp.where(low, jnp.minimum(t, p), jnp.maximum(t, p))
            else:
                low = (lane & j) == 0
                pu = pltpu.roll(t, 128 - j, axis=1)
                pd = pltpu.roll(t, j, axis=1)
                t = jnp.where(low, jnp.minimum(t, pu), jnp.maximum(t, pd))
            j //= 2
        if s is not None:
            t = t * s
        k *= 2
    return t.reshape(r, n)


def _flip_rows(v):
    r, n = v.shape
    cc = n // 128
    v3 = v.reshape(r, cc, 128)
    v3 = jnp.concatenate([v3[:, i : i + 1, :] for i in range(cc - 1, -1, -1)], axis=1)
    lane = jax.lax.broadcasted_iota(jnp.int32, (r, cc, 128), 2)
    out = v3
    j = 64
    while j >= 1:
        low = (lane & j) == 0
        out = jnp.where(low, pltpu.roll(out, 128 - j, axis=2), pltpu.roll(out, j, axis=2))
        j //= 2
    return out.reshape(r, n)


def _bf16_dot(a, b_t):
    return jax.lax.dot_general(
        a.astype(jnp.bfloat16),
        b_t.astype(jnp.bfloat16),
        (((1,), (1,)), ((), ())),
        preferred_element_type=jnp.float32,
    )


def _fused_kernel(x_ref, w1_ref, b1_ref, w2_ref, b2_ref, ws_ref, bs_ref, c_ref,
                  out_ref, lg_ref):
    x = x_ref[...]
    r = x.shape[0]
    d = x.shape[1]

    h = _bf16_dot(x, w1_ref[...]) + b1_ref[...]
    h = jnp.maximum(h, 0.0)
    enc = _bf16_dot(h, w2_ref[...]) + b2_ref[...]
    lg = _bf16_dot(enc, ws_ref[...]) + bs_ref[...]
    lg_ref[...] = lg

    add_s = c_ref[0, 0]
    sub_s = c_ref[0, 1]
    mul_s = c_ref[0, 2]
    div_s = c_ref[0, 3]
    inv_div = 1.0 / (div_s + 1e-05)

    col = jax.lax.broadcasted_iota(jnp.int32, (r, _S * _M), 1)
    a = jnp.ones((r, 1), jnp.float32)
    b = jnp.zeros((r, 1), jnp.float32)
    srt = jnp.zeros((r, 1), jnp.float32)
    flp = jnp.zeros((r, 1), jnp.float32)
    step_abs = []
    for s in range(_S):
        mask = (col >= s * _M) & (col < (s + 1) * _M)
        ml = jnp.where(mask, lg, -jnp.inf)
        mx = jnp.max(ml, axis=1, keepdims=True)
        idx = jnp.min(
            jnp.where(mask & (lg == mx), col - s * _M, _M), axis=1, keepdims=True
        )
        idx_f = idx.astype(jnp.float32)
        ind = [jnp.maximum(1.0 - jnp.abs(idx_f - k), 0.0) for k in range(_M)]
        a_op = 1.0 + ind[4] * (mul_s - 1.0) + ind[5] * (inv_div - 1.0)
        b_op = ind[2] * add_s - ind[3] * sub_s
        a_neg = jnp.maximum(jnp.sign(-a), 0.0)
        flp = (ind[0] * a_neg + ind[1] * (1.0 - flp)
               + (1.0 - ind[0] - ind[1]) * flp)
        srt = jnp.maximum(srt, ind[0])
        a = a * a_op
        b = b * a_op + b_op
        step_abs.append((a, b, srt, flp))

    sx = _bitonic_sort_rows(x)
    fx = _flip_rows(x)
    fsx = _flip_rows(sx)

    for s in range(_S):
        a, b, srt, flp = step_abs[s]
        base = (srt * (flp * fsx + (1.0 - flp) * sx)
                + (1.0 - srt) * (flp * fx + (1.0 - flp) * x))
        out_ref[:, s * d : (s + 1) * d] = a * base + b


@jax.jit
def kernel(x, W1, b1, W2, b2, Ws, bs, add_s, sub_s, mul_s, div_s):
    B, D = x.shape
    H = W1.shape[0]
    BR = 256
    consts = jnp.stack([add_s, sub_s, mul_s, div_s]).reshape(1, 4).astype(jnp.float32)
    out, lg = pl.pallas_call(
        _fused_kernel,
        grid=(B // BR,),
        in_specs=[
            pl.BlockSpec((BR, D), lambda i: (i, 0)),
            pl.BlockSpec((H, D), lambda i: (0, 0)),
            pl.BlockSpec((1, H), lambda i: (0, 0)),
            pl.BlockSpec((H, H), lambda i: (0, 0)),
            pl.BlockSpec((1, H), lambda i: (0, 0)),
            pl.BlockSpec((_S * _M, H), lambda i: (0, 0)),
            pl.BlockSpec((1, _S * _M), lambda i: (0, 0)),
            pl.BlockSpec((1, 4), lambda i: (0, 0)),
        ],
        out_specs=[
            pl.BlockSpec((BR, _S * D), lambda i: (i, 0)),
            pl.BlockSpec((BR, _S * _M), lambda i: (i, 0)),
        ],
        out_shape=[
            jax.ShapeDtypeStruct((B, _S * D), jnp.float32),
            jax.ShapeDtypeStruct((B, _S * _M), jnp.float32),
        ],
        compiler_params=pltpu.CompilerParams(
            dimension_semantics=("arbitrary",),
        ),
    )(x, W1, b1.reshape(1, H), W2, b2.reshape(1, H), Ws, bs.reshape(1, _S * _M),
      consts)
    return out.reshape(B, _S, D), lg.reshape(B, _S, _M)


# speedup vs baseline: 8.9703x; 1.6984x over previous
import functools

import jax
import jax.numpy as jnp
from jax.experimental import pallas as pl
from jax.experimental.pallas import tpu as pltpu

_S = 4
_M = 6


def _sort_cols(t):
    n, rr = t.shape
    i0 = jax.lax.broadcasted_iota(jnp.int32, (n, rr), 0)
    k = 2
    while k <= n:
        s = None
        if k < n:
            bit = i0 & k
            s = 1.0 - bit.astype(jnp.float32) * (2.0 / k)
            t = t * s
        j = k // 2
        while j >= 1:
            if j >= 8:
                t5 = t.reshape(n // (2 * j), 2, j, rr)
                a = t5[:, 0]
                b = t5[:, 1]
                t = jnp.concatenate(
                    [jnp.minimum(a, b)[:, None], jnp.maximum(a, b)[:, None]],
                    axis=1,
                ).reshape(n, rr)
            else:
                low = (i0 & j) == 0
                pu = pltpu.roll(t, n - j, axis=0)
                pd = pltpu.roll(t, j, axis=0)
                t = jnp.where(low, jnp.minimum(t, pu), jnp.maximum(t, pd))
            j //= 2
        if s is not None:
            t = t * s
        k *= 2
    return t


def _flip_cols(v):
    n, rr = v.shape
    i0 = jax.lax.broadcasted_iota(jnp.int32, (n, rr), 0)
    out = v
    j = n // 2
    while j >= 1:
        if j >= 8:
            v5 = out.reshape(n // (2 * j), 2, j, rr)
            out = jnp.concatenate([v5[:, 1:2], v5[:, 0:1]], axis=1).reshape(n, rr)
        else:
            low = (i0 & j) == 0
            out = jnp.where(
                low,
                pltpu.roll(out, n - j, axis=0),
                pltpu.roll(out, j, axis=0),
            )
        j //= 2
    return out


def _bf16_dot(w, xt):
    return jax.lax.dot_general(
        w.astype(jnp.bfloat16),
        xt.astype(jnp.bfloat16),
        (((1,), (0,)), ((), ())),
        preferred_element_type=jnp.float32,
    )


def _fused_kernel(xt_ref, w1_ref, b1_ref, w2_ref, b2_ref, ws_ref, bs_ref, c_ref,
                  out_ref, lg_ref):
    xt = xt_ref[...]
    d, r = xt.shape

    h = _bf16_dot(w1_ref[...], xt) + b1_ref[...]
    h = jnp.maximum(h, 0.0)
    enc = _bf16_dot(w2_ref[...], h) + b2_ref[...]
    lg = _bf16_dot(ws_ref[...], enc) + bs_ref[...]
    lg_ref[...] = lg

    add_s = c_ref[0, 0]
    sub_s = c_ref[0, 1]
    mul_s = c_ref[0, 2]
    div_s = c_ref[0, 3]
    inv_div = 1.0 / (div_s + 1e-05)

    a = jnp.ones((1, r), jnp.float32)
    b = jnp.zeros((1, r), jnp.float32)
    srt = jnp.zeros((1, r), jnp.float32)
    flp = jnp.zeros((1, r), jnp.float32)
    i6 = jax.lax.broadcasted_iota(jnp.int32, (_M, r), 0)
    step_abs = []
    for s in range(_S):
        sl = lg[s * _M:(s + 1) * _M, :]
        mx = jnp.max(sl, axis=0, keepdims=True)
        idx = jnp.min(
            jnp.where(sl == mx, i6, _M), axis=0, keepdims=True
        )
        idx_f = idx.astype(jnp.float32)
        ind = [jnp.maximum(1.0 - jnp.abs(idx_f - k), 0.0) for k in range(_M)]
        a_op = 1.0 + ind[4] * (mul_s - 1.0) + ind[5] * (inv_div - 1.0)
        b_op = ind[2] * add_s - ind[3] * sub_s
        a_neg = jnp.maximum(jnp.sign(-a), 0.0)
        flp = (ind[0] * a_neg + ind[1] * (1.0 - flp)
               + (1.0 - ind[0] - ind[1]) * flp)
        srt = jnp.maximum(srt, ind[0])
        a = a * a_op
        b = b * a_op + b_op
        step_abs.append((a, b, srt, flp))

    sx = _sort_cols(xt)
    fx = _flip_cols(xt)
    fsx = _flip_cols(sx)

    for s in range(_S):
        a, b, srt, flp = step_abs[s]
        base = (srt * (flp * fsx + (1.0 - flp) * sx)
                + (1.0 - srt) * (flp * fx + (1.0 - flp) * xt))
        out_ref[s * d:(s + 1) * d, :] = a * base + b


@jax.jit
def kernel(x, W1, b1, W2, b2, Ws, bs, add_s, sub_s, mul_s, div_s):
    B, D = x.shape
    H = W1.shape[0]
    BR = 256
    xt = x.T
    consts = jnp.stack([add_s, sub_s, mul_s, div_s]).reshape(1, 4).astype(jnp.float32)
    out_t, lg_t = pl.pallas_call(
        _fused_kernel,
        grid=(B // BR,),
        in_specs=[
            pl.BlockSpec((D, BR), lambda i: (0, i)),
            pl.BlockSpec((H, D), lambda i: (0, 0)),
            pl.BlockSpec((H, 1), lambda i: (0, 0)),
            pl.BlockSpec((H, H), lambda i: (0, 0)),
            pl.BlockSpec((H, 1), lambda i: (0, 0)),
            pl.BlockSpec((_S * _M, H), lambda i: (0, 0)),
            pl.BlockSpec((_S * _M, 1), lambda i: (0, 0)),
            pl.BlockSpec((1, 4), lambda i: (0, 0)),
        ],
        out_specs=[
            pl.BlockSpec((_S * D, BR), lambda i: (0, i)),
            pl.BlockSpec((_S * _M, BR), lambda i: (0, i)),
        ],
        out_shape=[
            jax.ShapeDtypeStruct((_S * D, B), jnp.float32),
            jax.ShapeDtypeStruct((_S * _M, B), jnp.float32),
        ],
        compiler_params=pltpu.CompilerParams(
            dimension_semantics=("arbitrary",),
        ),
    )(xt, W1, b1.reshape(H, 1), W2, b2.reshape(H, 1), Ws,
      bs.reshape(_S * _M, 1), consts)
    out = out_t.reshape(_S, D, B).transpose(2, 0, 1)
    lg = lg_t.T.reshape(B, _S, _M)
    return out, lg


# speedup vs baseline: 11.2028x; 1.2489x over previous
import functools

import jax
import jax.numpy as jnp
from jax.experimental import pallas as pl
from jax.experimental.pallas import tpu as pltpu

_S = 4
_M = 6


def _xor_shuffle(t, p0, dist):
    n, rr = t.shape
    if dist >= 8:
        t5 = t.reshape(n // (2 * dist), 2, dist, rr)
        return jnp.concatenate([t5[:, 1:2], t5[:, 0:1]], axis=1).reshape(n, rr)
    low = (p0 & dist) == 0
    return jnp.where(
        low, pltpu.roll(t, n - dist, axis=0), pltpu.roll(t, dist, axis=0)
    )


def _sort_cols_swz(t, p0, i_idx):
    n, rr = t.shape
    k = 2
    while k <= n:
        s = None
        if k < n:
            bit = i_idx & k
            s = 1.0 - bit.astype(jnp.float32) * (2.0 / k)
            t = t * s
        j = k // 2
        while j >= 1:
            if j <= 128:
                jp = 8 * j
                t5 = t.reshape(n // (2 * jp), 2, jp, rr)
                a = t5[:, 0]
                b = t5[:, 1]
                t = jnp.concatenate(
                    [jnp.minimum(a, b)[:, None], jnp.maximum(a, b)[:, None]],
                    axis=1,
                ).reshape(n, rr)
            else:
                jr = j // 256
                low = (p0 & jr) == 0
                pu = pltpu.roll(t, n - jr, axis=0)
                pd = pltpu.roll(t, jr, axis=0)
                t = jnp.where(low, jnp.minimum(t, pu), jnp.maximum(t, pd))
            j //= 2
        if s is not None:
            t = t * s
        k *= 2
    return t


def _flip_cols_swz(t, p0):
    for dist in (1024, 512, 256, 128, 64, 32, 16, 8, 4, 2, 1):
        t = _xor_shuffle(t, p0, dist)
    return t


def _bf16_dot(w, xt):
    return jax.lax.dot_general(
        w.astype(jnp.bfloat16),
        xt.astype(jnp.bfloat16),
        (((1,), (0,)), ((), ())),
        preferred_element_type=jnp.float32,
    )


def _fused_kernel(xs_ref, w1_ref, b1_ref, w2_ref, b2_ref, ws_ref, bs_ref, c_ref,
                  out_ref, lg_ref):
    xs = xs_ref[...]
    d, r = xs.shape

    h = _bf16_dot(w1_ref[...], xs) + b1_ref[...]
    h = jnp.maximum(h, 0.0)
    enc = _bf16_dot(w2_ref[...], h) + b2_ref[...]
    lg = _bf16_dot(ws_ref[...], enc) + bs_ref[...]
    lg_ref[...] = lg

    add_s = c_ref[0, 0]
    sub_s = c_ref[0, 1]
    mul_s = c_ref[0, 2]
    div_s = c_ref[0, 3]
    inv_div = 1.0 / (div_s + 1e-05)

    a = jnp.ones((1, r), jnp.float32)
    b = jnp.zeros((1, r), jnp.float32)
    srt = jnp.zeros((1, r), jnp.float32)
    flp = jnp.zeros((1, r), jnp.float32)
    i6 = jax.lax.broadcasted_iota(jnp.int32, (_M, r), 0)
    step_abs = []
    for s in range(_S):
        sl = lg[s * _M:(s + 1) * _M, :]
        mx = jnp.max(sl, axis=0, keepdims=True)
        idx = jnp.min(
            jnp.where(sl == mx, i6, _M), axis=0, keepdims=True
        )
        idx_f = idx.astype(jnp.float32)
        ind = [jnp.maximum(1.0 - jnp.abs(idx_f - k), 0.0) for k in range(_M)]
        a_op = 1.0 + ind[4] * (mul_s - 1.0) + ind[5] * (inv_div - 1.0)
        b_op = ind[2] * add_s - ind[3] * sub_s
        a_neg = jnp.maximum(jnp.sign(-a), 0.0)
        flp = (ind[0] * a_neg + ind[1] * (1.0 - flp)
               + (1.0 - ind[0] - ind[1]) * flp)
        srt = jnp.maximum(srt, ind[0])
        a = a * a_op
        b = b * a_op + b_op
        step_abs.append((a, b, srt, flp))

    p0 = jax.lax.broadcasted_iota(jnp.int32, (d, r), 0)
    i_idx = (p0 & 7) * 256 + (p0 >> 3)
    sx = _sort_cols_swz(xs, p0, i_idx)
    fx = _flip_cols_swz(xs, p0)
    fsx = _flip_cols_swz(sx, p0)

    for s in range(_S):
        a, b, srt, flp = step_abs[s]
        base = (srt * (flp * fsx + (1.0 - flp) * sx)
                + (1.0 - srt) * (flp * fx + (1.0 - flp) * xs))
        out_ref[s * d:(s + 1) * d, :] = a * base + b


@jax.jit
def kernel(x, W1, b1, W2, b2, Ws, bs, add_s, sub_s, mul_s, div_s):
    B, D = x.shape
    H = W1.shape[0]
    BR = 256
    xs = x.T.reshape(8, 256, B).transpose(1, 0, 2).reshape(D, B)
    W1s = W1.reshape(H, 8, 256).transpose(0, 2, 1).reshape(H, D)
    consts = jnp.stack([add_s, sub_s, mul_s, div_s]).reshape(1, 4).astype(jnp.float32)
    out_t, lg_t = pl.pallas_call(
        _fused_kernel,
        grid=(B // BR,),
        in_specs=[
            pl.BlockSpec((D, BR), lambda i: (0, i)),
            pl.BlockSpec((H, D), lambda i: (0, 0)),
            pl.BlockSpec((H, 1), lambda i: (0, 0)),
            pl.BlockSpec((H, H), lambda i: (0, 0)),
            pl.BlockSpec((H, 1), lambda i: (0, 0)),
            pl.BlockSpec((_S * _M, H), lambda i: (0, 0)),
            pl.BlockSpec((_S * _M, 1), lambda i: (0, 0)),
            pl.BlockSpec((1, 4), lambda i: (0, 0)),
        ],
        out_specs=[
            pl.BlockSpec((_S * D, BR), lambda i: (0, i)),
            pl.BlockSpec((_S * _M, BR), lambda i: (0, i)),
        ],
        out_shape=[
            jax.ShapeDtypeStruct((_S * D, B), jnp.float32),
            jax.ShapeDtypeStruct((_S * _M, B), jnp.float32),
        ],
        compiler_params=pltpu.CompilerParams(
            dimension_semantics=("arbitrary",),
        ),
    )(xs, W1s, b1.reshape(H, 1), W2, b2.reshape(H, 1), Ws,
      bs.reshape(_S * _M, 1), consts)
    out = out_t.reshape(_S, 256, 8, B).transpose(3, 0, 2, 1).reshape(B, _S, D)
    lg = lg_t.T.reshape(B, _S, _M)
    return out, lg


# speedup vs baseline: 11.2158x; 1.0012x over previous
import functools

import jax
import jax.numpy as jnp
from jax.experimental import pallas as pl
from jax.experimental.pallas import tpu as pltpu

_S = 4
_M = 6


def _xor_shuffle(t, p0, dist):
    n, rr = t.shape
    if dist >= 8:
        t5 = t.reshape(n // (2 * dist), 2, dist, rr)
        return jnp.concatenate([t5[:, 1:2], t5[:, 0:1]], axis=1).reshape(n, rr)
    low = (p0 & dist) == 0
    return jnp.where(
        low, pltpu.roll(t, n - dist, axis=0), pltpu.roll(t, dist, axis=0)
    )


def _sort_cols_swz(t, p0, i_idx):
    n, rr = t.shape
    k = 2
    while k <= n:
        s = None
        if k < n:
            bit = i_idx & k
            s = 1.0 - bit.astype(jnp.float32) * (2.0 / k)
            t = t * s
        j = k // 2
        while j >= 1:
            if j <= 128:
                jp = 8 * j
                t5 = t.reshape(n // (2 * jp), 2, jp, rr)
                a = t5[:, 0]
                b = t5[:, 1]
                t = jnp.concatenate(
                    [jnp.minimum(a, b)[:, None], jnp.maximum(a, b)[:, None]],
                    axis=1,
                ).reshape(n, rr)
            else:
                jr = j // 256
                low = (p0 & jr) == 0
                pu = pltpu.roll(t, n - jr, axis=0)
                pd = pltpu.roll(t, jr, axis=0)
                t = jnp.where(low, jnp.minimum(t, pu), jnp.maximum(t, pd))
            j //= 2
        if s is not None:
            t = t * s
        k *= 2
    return t


def _flip_cols_swz(t, p0):
    for dist in (1024, 512, 256, 128, 64, 32, 16, 8, 4, 2, 1):
        t = _xor_shuffle(t, p0, dist)
    return t


def _bf16_dot(w, xt):
    return jax.lax.dot_general(
        w.astype(jnp.bfloat16),
        xt.astype(jnp.bfloat16),
        (((1,), (0,)), ((), ())),
        preferred_element_type=jnp.float32,
    )


def _fused_kernel(xs_ref, w1_ref, b1_ref, w2_ref, b2_ref, ws_ref, bs_ref, c_ref,
                  out_ref, lg_ref):
    xs = xs_ref[...]
    d, r = xs.shape

    h = _bf16_dot(w1_ref[...], xs) + b1_ref[...]
    h = jnp.maximum(h, 0.0)
    enc = _bf16_dot(w2_ref[...], h) + b2_ref[...]
    lg = _bf16_dot(ws_ref[...], enc) + bs_ref[...]
    lg_ref[...] = lg

    add_s = c_ref[0, 0]
    sub_s = c_ref[0, 1]
    mul_s = c_ref[0, 2]
    div_s = c_ref[0, 3]
    inv_div = 1.0 / (div_s + 1e-05)

    a = jnp.ones((1, r), jnp.float32)
    b = jnp.zeros((1, r), jnp.float32)
    srt = jnp.zeros((1, r), jnp.float32)
    flp = jnp.zeros((1, r), jnp.float32)
    i6 = jax.lax.broadcasted_iota(jnp.int32, (_M, r), 0)
    step_abs = []
    for s in range(_S):
        sl = lg[s * _M:(s + 1) * _M, :]
        mx = jnp.max(sl, axis=0, keepdims=True)
        idx = jnp.min(
            jnp.where(sl == mx, i6, _M), axis=0, keepdims=True
        )
        idx_f = idx.astype(jnp.float32)
        ind = [jnp.maximum(1.0 - jnp.abs(idx_f - k), 0.0) for k in range(_M)]
        a_op = 1.0 + ind[4] * (mul_s - 1.0) + ind[5] * (inv_div - 1.0)
        b_op = ind[2] * add_s - ind[3] * sub_s
        a_neg = jnp.maximum(jnp.sign(-a), 0.0)
        flp = (ind[0] * a_neg + ind[1] * (1.0 - flp)
               + (1.0 - ind[0] - ind[1]) * flp)
        srt = jnp.maximum(srt, ind[0])
        a = a * a_op
        b = b * a_op + b_op
        step_abs.append((a, b, srt, flp))

    p0 = jax.lax.broadcasted_iota(jnp.int32, (d, r), 0)
    i_idx = (p0 & 7) * 256 + (p0 >> 3)
    sx = _sort_cols_swz(xs, p0, i_idx)
    fx = _flip_cols_swz(xs, p0)
    fsx = _flip_cols_swz(sx, p0)

    for s in range(_S):
        a, b, srt, flp = step_abs[s]
        base = (srt * (flp * fsx + (1.0 - flp) * sx)
                + (1.0 - srt) * (flp * fx + (1.0 - flp) * xs))
        out_ref[s * d:(s + 1) * d, :] = a * base + b


@jax.jit
def kernel(x, W1, b1, W2, b2, Ws, bs, add_s, sub_s, mul_s, div_s):
    B, D = x.shape
    H = W1.shape[0]
    BR = 256
    xs = x.T.reshape(8, 256, B).transpose(1, 0, 2).reshape(D, B)
    W1s = W1.reshape(H, 8, 256).transpose(0, 2, 1).reshape(H, D)
    consts = jnp.stack([add_s, sub_s, mul_s, div_s]).reshape(1, 4).astype(jnp.float32)
    out_t, lg_t = pl.pallas_call(
        _fused_kernel,
        grid=(B // BR,),
        in_specs=[
            pl.BlockSpec((D, BR), lambda i: (0, i)),
            pl.BlockSpec((H, D), lambda i: (0, 0)),
            pl.BlockSpec((H, 1), lambda i: (0, 0)),
            pl.BlockSpec((H, H), lambda i: (0, 0)),
            pl.BlockSpec((H, 1), lambda i: (0, 0)),
            pl.BlockSpec((_S * _M, H), lambda i: (0, 0)),
            pl.BlockSpec((_S * _M, 1), lambda i: (0, 0)),
            pl.BlockSpec((1, 4), lambda i: (0, 0)),
        ],
        out_specs=[
            pl.BlockSpec((_S * D, BR), lambda i: (0, i)),
            pl.BlockSpec((_S * _M, BR), lambda i: (0, i)),
        ],
        out_shape=[
            jax.ShapeDtypeStruct((_S * D, B), jnp.float32),
            jax.ShapeDtypeStruct((_S * _M, B), jnp.float32),
        ],
        compiler_params=pltpu.CompilerParams(
            dimension_semantics=("parallel",),
        ),
    )(xs, W1s, b1.reshape(H, 1), W2, b2.reshape(H, 1), Ws,
      bs.reshape(_S * _M, 1), consts)
    out = out_t.reshape(_S, 256, 8, B).transpose(3, 0, 2, 1).reshape(B, _S, D)
    lg = lg_t.T.reshape(B, _S, _M)
    return out, lg
